# baseline (speedup 1.0000x reference)
import functools

import jax
import jax.numpy as jnp
from jax import lax
from jax.experimental import pallas as pl
from jax.experimental.pallas import tpu as pltpu
from jax.experimental.pallas import tpu_sc as plsc

B = 16384
D = 64
HIDDEN = 100
NC = 2
NS = 16
NW = NC * NS
BPW = B // NW
CH = 128
NCH = BPW // CH
L = 16


def _sc_gather(x1, x2, x3, user_pairs, movie_pairs, category_pairs):
    mesh = plsc.VectorSubcoreMesh(core_axis_name="c", subcore_axis_name="s")

    @functools.partial(
        pl.kernel,
        mesh=mesh,
        out_type=[jax.ShapeDtypeStruct((B, 2 * D), jnp.float32)] * 3,
        scratch_types=[
            pltpu.VMEM((BPW,), jnp.int32),
            pltpu.VMEM((BPW,), jnp.int32),
            pltpu.VMEM((BPW, 2 * D), jnp.float32),
            pltpu.SemaphoreType.DMA,
            pltpu.SemaphoreType.DMA,
        ],
    )
    def gather_kernel(x1h, x2h, x3h, uh, mh, ch, o1h, o2h, o3h,
                      iraw, ipair, rows, gsem, wsem):
        wid = lax.axis_index("s") * NC + lax.axis_index("c")
        base = wid * BPW

        for xh, tbl, oh in ((x1h, uh, o1h), (x2h, mh, o2h), (x3h, ch, o3h)):
            pltpu.sync_copy(xh.at[pl.ds(base, BPW)], iraw)

            def halve(k, _):
                sl = pl.ds(k * L, L)
                ipair[sl] = lax.shift_right_logical(iraw[sl], 1)
                return _

            lax.fori_loop(0, BPW // L, halve, 0)

            gcps = []
            for j in range(NCH):
                sl = pl.ds(j * CH, CH)
                gcps.append(
                    pltpu.async_copy(tbl.at[ipair.at[sl]], rows.at[sl], gsem))
            wcps = []
            for j in range(NCH):
                sl = pl.ds(j * CH, CH)
                gcps[j].wait()
                wcps.append(
                    pltpu.async_copy(
                        rows.at[sl], oh.at[pl.ds(base + j * CH, CH)], wsem))
            for w in wcps:
                w.wait()

    return gather_kernel(x1, x2, x3, user_pairs, movie_pairs, category_pairs)


RB = 2048


def _mlp_kernel(x1r, x2r, x3r, e1r, e2r, e3r, w1r, b1r, w2r, b2r, outr):
    def half(xr, er):
        odd = (xr[...] & 1) == 1
        return jnp.where(odd, er[:, D:2 * D], er[:, 0:D])

    h = jnp.dot(half(x1r, e1r), w1r[0:D, :], preferred_element_type=jnp.float32)
    h = h + jnp.dot(half(x2r, e2r), w1r[D:2 * D, :],
                    preferred_element_type=jnp.float32)
    h = h + jnp.dot(half(x3r, e3r), w1r[2 * D:3 * D, :],
                    preferred_element_type=jnp.float32)
    h = jnp.maximum(h + b1r[...], 0.0)
    o = jnp.dot(h, w2r[...], preferred_element_type=jnp.float32) + b2r[...]
    outr[...] = 1.0 / (1.0 + jnp.exp(-o))


def _mlp(x1, x2, x3, e1, e2, e3, W1, b1, W2, b2):
    grid = (B // RB,)
    return pl.pallas_call(
        _mlp_kernel,
        grid=grid,
        in_specs=[
            pl.BlockSpec((RB, 1), lambda i: (i, 0)),
            pl.BlockSpec((RB, 1), lambda i: (i, 0)),
            pl.BlockSpec((RB, 1), lambda i: (i, 0)),
            pl.BlockSpec((RB, 2 * D), lambda i: (i, 0)),
            pl.BlockSpec((RB, 2 * D), lambda i: (i, 0)),
            pl.BlockSpec((RB, 2 * D), lambda i: (i, 0)),
            pl.BlockSpec((3 * D, HIDDEN), lambda i: (0, 0)),
            pl.BlockSpec((1, HIDDEN), lambda i: (0, 0)),
            pl.BlockSpec((HIDDEN, 1), lambda i: (0, 0)),
            pl.BlockSpec((1, 1), lambda i: (0, 0)),
        ],
        out_specs=pl.BlockSpec((RB, 1), lambda i: (i, 0)),
        out_shape=jax.ShapeDtypeStruct((B, 1), jnp.float32),
    )(x1, x2, x3, e1, e2, e3, W1, b1, W2, b2)


def kernel(x1, x2, x3, user_embed, movie_embed, category_embed, W1, b1, W2, b2):
    x1 = x1.astype(jnp.int32)
    x2 = x2.astype(jnp.int32)
    x3 = x3.astype(jnp.int32)
    e1, e2, e3 = _sc_gather(
        x1, x2, x3,
        user_embed.reshape(-1, 2 * D),
        movie_embed.reshape(-1, 2 * D),
        category_embed.reshape(-1, 2 * D),
    )
    return _mlp(x1.reshape(B, 1), x2.reshape(B, 1), x3.reshape(B, 1),
                e1, e2, e3, W1, b1.reshape(1, HIDDEN), W2, b2.reshape(1, 1))


# speedup vs baseline: 1.9144x; 1.9144x over previous
import functools

import jax
import jax.numpy as jnp
from jax import lax
from jax.experimental import pallas as pl
from jax.experimental.pallas import tpu as pltpu
from jax.experimental.pallas import tpu_sc as plsc

B = 16384
D = 64
HIDDEN = 100
NC = 2
NS = 16
NW = NC * NS
BPW = B // NW
PPW = BPW // 2
L = 16
TR = 8
CHB = 32
NCHB = BPW // CHB


def _sc_gather(x1, x2, x3, user_t, movie_t, category_t):
    mesh = plsc.VectorSubcoreMesh(core_axis_name="c", subcore_axis_name="s")

    @functools.partial(
        pl.kernel,
        mesh=mesh,
        out_type=[jax.ShapeDtypeStruct((B // 2, 2 * D), jnp.float32)] * 3,
        scratch_types=[
            pltpu.VMEM((BPW,), jnp.int32),
            pltpu.VMEM((2, CHB, TR, D), jnp.float32),
            pltpu.VMEM((PPW, 2 * D), jnp.float32),
            pltpu.SemaphoreType.DMA,
        ],
    )
    def gather_kernel(x1h, x2h, x3h, uh, mh, ch, o1h, o2h, o3h,
                      iv, gbuf, stag, gsem):
        wid = lax.axis_index("s") * NC + lax.axis_index("c")
        base = wid * BPW
        obase = wid * PPW

        for xh, tbl, oh in ((x1h, uh, o1h), (x2h, mh, o2h), (x3h, ch, o3h)):
            pltpu.sync_copy(xh.at[pl.ds(base, BPW)], iv)

            def fire(t, slot):
                def grp(gi, _):
                    vec = iv[pl.ds(t * CHB + gi * L, L)]
                    for lane in range(L):
                        g = lax.shift_right_logical(vec[lane], 3)
                        pltpu.async_copy(
                            tbl.at[pl.ds(g, 1)],
                            gbuf.at[slot].at[pl.ds(gi * L + lane, 1)], gsem)
                    return _

                lax.fori_loop(0, CHB // L, grp, 0)

            def drain(slot):
                pltpu.make_async_copy(
                    tbl.at[pl.ds(0, CHB)], gbuf.at[slot], gsem).wait()

            def select(t, slot):
                gb = gbuf.at[slot]

                def sgrp(gi, _):
                    vec = iv[pl.ds(t * CHB + gi * L, L)]
                    for li in range(L // 2):
                        r0 = vec[2 * li] & (TR - 1)
                        r1 = vec[2 * li + 1] & (TR - 1)
                        p = t * (CHB // 2) + gi * (L // 2) + li
                        row = gi * L + 2 * li
                        for q in range(D // L):
                            sl = pl.ds(q * L, L)
                            stag[p, sl] = gb[row, r0, sl]
                            stag[p, pl.ds(D + q * L, L)] = gb[row + 1, r1, sl]
                    return _

                lax.fori_loop(0, CHB // L, sgrp, 0)

            fire(0, 0)

            def step(t, _):
                @pl.when(t + 1 < NCHB)
                def _prefetch():
                    fire(t + 1, lax.rem(t + 1, 2))

                slot = lax.rem(t, 2)
                drain(slot)
                select(t, slot)
                return _

            lax.fori_loop(0, NCHB, step, 0)
            pltpu.sync_copy(stag, oh.at[pl.ds(obase, PPW)])

    return gather_kernel(
        x1, x2, x3,
        user_t.reshape(-1, TR, D),
        movie_t.reshape(-1, TR, D),
        category_t.reshape(-1, TR, D),
    )


RB2 = 1024


def _mlp_kernel(e1r, e2r, e3r, w1r, b1r, w2r, b2r, outr):
    def head(lo):
        sl = slice(lo, lo + D)
        h = jnp.dot(e1r[:, sl], w1r[0:D, :], preferred_element_type=jnp.float32)
        h = h + jnp.dot(e2r[:, sl], w1r[D:2 * D, :],
                        preferred_element_type=jnp.float32)
        h = h + jnp.dot(e3r[:, sl], w1r[2 * D:3 * D, :],
                        preferred_element_type=jnp.float32)
        h = jnp.maximum(h + b1r[...], 0.0)
        o = jnp.dot(h, w2r[...], preferred_element_type=jnp.float32) + b2r[...]
        return 1.0 / (1.0 + jnp.exp(-o))

    outr[...] = jnp.concatenate([head(0), head(D)], axis=1)


def _mlp(e1, e2, e3, W1, b1, W2, b2):
    grid = (B // 2 // RB2,)
    return pl.pallas_call(
        _mlp_kernel,
        grid=grid,
        in_specs=[
            pl.BlockSpec((RB2, 2 * D), lambda i: (i, 0)),
            pl.BlockSpec((RB2, 2 * D), lambda i: (i, 0)),
            pl.BlockSpec((RB2, 2 * D), lambda i: (i, 0)),
            pl.BlockSpec((3 * D, HIDDEN), lambda i: (0, 0)),
            pl.BlockSpec((1, HIDDEN), lambda i: (0, 0)),
            pl.BlockSpec((HIDDEN, 1), lambda i: (0, 0)),
            pl.BlockSpec((1, 1), lambda i: (0, 0)),
        ],
        out_specs=pl.BlockSpec((RB2, 2), lambda i: (i, 0)),
        out_shape=jax.ShapeDtypeStruct((B // 2, 2), jnp.float32),
    )(e1, e2, e3, W1, b1, W2, b2)


def kernel(x1, x2, x3, user_embed, movie_embed, category_embed, W1, b1, W2, b2):
    x1 = x1.astype(jnp.int32)
    x2 = x2.astype(jnp.int32)
    x3 = x3.astype(jnp.int32)
    e1, e2, e3 = _sc_gather(
        x1, x2, x3, user_embed, movie_embed, category_embed)
    out = _mlp(e1, e2, e3, W1, b1.reshape(1, HIDDEN), W2, b2.reshape(1, 1))
    return out.reshape(B, 1)
